# baseline (speedup 1.0000x reference)
import functools

import jax
import jax.numpy as jnp
from jax import lax
from jax.experimental import pallas as pl
from jax.experimental.pallas import tpu as pltpu
from jax.experimental.pallas import tpu_sc as plsc

N = 10000
E = 320000
D = 128

NC = 2
NS = 16
NW = NC * NS
CH = 128
CPT = 84
EPT = CH * CPT
E_PAD = NW * EPT
EPTG_S = EPT + 4 * CH
EPTG_D = EPT + 2 * CH
NPAD = 10112
ROWS_PT = NPAD // NS
RB = 1000

_mesh = plsc.VectorSubcoreMesh(core_axis_name="c", subcore_axis_name="s")


@functools.partial(
    pl.kernel,
    out_type=jax.ShapeDtypeStruct((NW, NPAD), jnp.float32),
    mesh=_mesh,
    compiler_params=pltpu.CompilerParams(needs_layout_passes=False),
    scratch_types=[
        pltpu.VMEM((EPT,), jnp.int32),
        pltpu.VMEM((NPAD,), jnp.float32),
    ],
)
def _sc_hist(dst_hbm, out_hbm, idx_v, hist_v):
    c = lax.axis_index("c")
    s = lax.axis_index("s")
    wid = c * NS + s
    pltpu.sync_copy(dst_hbm.at[pl.ds(wid * EPTG_D, EPT)], idx_v)

    zeros16 = jnp.zeros((16,), jnp.float32)

    def zbody(i, carry):
        hist_v[pl.ds(i * 16, 16)] = zeros16
        return carry

    lax.fori_loop(0, NPAD // 16, zbody, 0)

    ones16 = jnp.ones((16,), jnp.float32)

    def body(i, carry):
        idx16 = idx_v[pl.ds(i * 16, 16)]
        plsc.addupdate_scatter(hist_v, [idx16], ones16)
        return carry

    lax.fori_loop(0, EPT // 16, body, 0)
    pltpu.sync_copy(hist_v, out_hbm.at[wid])


@functools.partial(
    pl.kernel,
    out_type=jax.ShapeDtypeStruct((NC, NPAD, D), jnp.float32),
    mesh=_mesh,
    compiler_params=pltpu.CompilerParams(needs_layout_passes=False),
    scratch_types=(
        [pltpu.VMEM((CH, D), jnp.float32) for _ in range(3)]
        + [pltpu.VMEM((2 * CH,), jnp.int32) for _ in range(2)]
        + [pltpu.VMEM((2 * CH,), jnp.int32)]
        + [pltpu.VMEM((CH,), jnp.int32)]
        + [pltpu.SemaphoreType.DMA for _ in range(6)]
        + [pltpu.VMEM_SHARED((NPAD, D), jnp.float32)]
    ),
)
def _sc_agg(src_hbm, dst_hbm, y_hbm, out_hbm, *refs):
    rows = list(refs[0:3])
    sbuf = list(refs[3:5])
    dbuf = refs[5]
    dcur = refs[6]
    gsem = list(refs[7:10])
    isp = list(refs[10:12])
    idp = refs[12]
    acc = refs[13]
    c = lax.axis_index("c")
    s = lax.axis_index("s")
    wid = c * NS + s

    zeros16 = jnp.zeros((16,), jnp.float32)

    def zbody(i, carry):
        for sub in range(D // 16):
            rows[0][i, pl.ds(sub * 16, 16)] = zeros16
        return carry

    lax.fori_loop(0, CH, zbody, 0)

    base = s * ROWS_PT
    for k in range(ROWS_PT // CH):
        pltpu.sync_copy(rows[0], acc.at[pl.ds(base + k * CH, CH)])
    rem = ROWS_PT % CH
    if rem:
        pltpu.sync_copy(
            rows[0].at[pl.ds(0, rem)],
            acc.at[pl.ds(base + (ROWS_PT // CH) * CH, rem)],
        )
    plsc.subcore_barrier()

    def sidx(p):
        return src_hbm.at[pl.ds(wid * EPTG_S + p * 2 * CH, 2 * CH)]

    def didx(p):
        return dst_hbm.at[pl.ds(wid * EPTG_D + p * 2 * CH, 2 * CH)]

    def gref(pe, half):
        return y_hbm.at[sbuf[pe].at[pl.ds(half * CH, CH)]]

    def slot(ci, b, half, pe):
        if half == 0:
            pltpu.make_async_copy(sidx(0), sbuf[(pe + 1) % 2], isp[(pe + 1) % 2]).wait()
            pltpu.make_async_copy(didx(0), dbuf, idp).wait()
        pltpu.async_copy(gref((pe + 1) % 2, half), rows[(b + 2) % 3], gsem[(b + 2) % 3])
        pltpu.make_async_copy(gref(pe, half), rows[b], gsem[b]).wait()
        for sub in range(CH // 16):
            dcur[pl.ds(sub * 16, 16)] = dbuf[pl.ds(half * CH + sub * 16, 16)]
        if half == 1:
            e = ci // 2
            pltpu.async_copy(sidx(e + 2), sbuf[pe], isp[pe])
            pltpu.async_copy(didx(e + 1), dbuf, idp)
        pltpu.sync_copy(rows[b], acc.at[dcur], add=True)

    pltpu.async_copy(sidx(0), sbuf[0], isp[0])
    pltpu.async_copy(sidx(1), sbuf[1], isp[1])
    pltpu.async_copy(didx(0), dbuf, idp)
    pltpu.make_async_copy(sidx(0), sbuf[0], isp[0]).wait()
    pltpu.async_copy(gref(0, 0), rows[0], gsem[0])
    pltpu.async_copy(gref(0, 1), rows[1], gsem[1])

    def body(j, carry):
        for k in range(12):
            ci = 12 * j + k
            slot(ci, k % 3, k % 2, (k // 2) % 2)
        return carry

    lax.fori_loop(0, CPT // 12, body, 0)

    pltpu.make_async_copy(gref(0, 0), rows[0], gsem[0]).wait()
    pltpu.make_async_copy(gref(0, 1), rows[1], gsem[1]).wait()
    pltpu.make_async_copy(sidx(CPT // 2 + 1), sbuf[1], isp[1]).wait()
    pltpu.make_async_copy(didx(CPT // 2), dbuf, idp).wait()

    plsc.subcore_barrier()
    for k in range(ROWS_PT // CH):
        pltpu.sync_copy(acc.at[pl.ds(base + k * CH, CH)],
                        out_hbm.at[c, pl.ds(base + k * CH, CH)])
    if rem:
        r0 = base + (ROWS_PT // CH) * CH
        pltpu.sync_copy(acc.at[pl.ds(r0, rem)], out_hbm.at[c, pl.ds(r0, rem)])


def _tc_dinv(hist):
    def k(h_ref, o_ref):
        deg = jnp.sum(h_ref[...], axis=0, keepdims=True) + 1.0
        o_ref[...] = lax.rsqrt(deg)

    return pl.pallas_call(
        k, out_shape=jax.ShapeDtypeStruct((1, NPAD), jnp.float32)
    )(hist)


def _tc_scale(dinv_col, x):
    def k(d_ref, x_ref, o_ref):
        o_ref[...] = d_ref[...] * x_ref[...]

    return pl.pallas_call(
        k,
        grid=(N // RB,),
        in_specs=[
            pl.BlockSpec((RB, 1), lambda i: (i, 0)),
            pl.BlockSpec((RB, D), lambda i: (i, 0)),
        ],
        out_specs=pl.BlockSpec((RB, D), lambda i: (i, 0)),
        out_shape=jax.ShapeDtypeStruct((N, D), jnp.float32),
    )(dinv_col, x)


def _tc_mid(a0, a1, y, dinv_col, W1, b1, W2):
    def k(a0r, a1r, yr, dr, w1r, b1r, w2r, outr):
        s1 = dr[...] * (a0r[...] + a1r[...] + yr[...])
        x1 = jnp.dot(s1, w1r[...], preferred_element_type=jnp.float32) + b1r[...]
        z = jnp.maximum(x1, 0.0)
        outr[...] = dr[...] * jnp.dot(z, w2r[...], preferred_element_type=jnp.float32)

    row = lambda i: (i, 0)
    fix = lambda i: (0, 0)
    return pl.pallas_call(
        k,
        grid=(N // RB,),
        in_specs=[
            pl.BlockSpec((RB, D), row),
            pl.BlockSpec((RB, D), row),
            pl.BlockSpec((RB, D), row),
            pl.BlockSpec((RB, 1), row),
            pl.BlockSpec((D, 4 * D), fix),
            pl.BlockSpec((4 * D,), lambda i: (0,)),
            pl.BlockSpec((4 * D, D), fix),
        ],
        out_specs=pl.BlockSpec((RB, D), row),
        out_shape=jax.ShapeDtypeStruct((N, D), jnp.float32),
    )(a0, a1, y, dinv_col, W1, b1, W2)


def _tc_final(a0, a1, y2, dinv_col, b2):
    def k(a0r, a1r, yr, dr, b2r, outr):
        outr[...] = dr[...] * (a0r[...] + a1r[...] + yr[...]) + b2r[...]

    row = lambda i: (i, 0)
    return pl.pallas_call(
        k,
        grid=(N // RB,),
        in_specs=[
            pl.BlockSpec((RB, D), row),
            pl.BlockSpec((RB, D), row),
            pl.BlockSpec((RB, D), row),
            pl.BlockSpec((RB, 1), row),
            pl.BlockSpec((D,), lambda i: (0,)),
        ],
        out_specs=pl.BlockSpec((RB, D), row),
        out_shape=jax.ShapeDtypeStruct((N, D), jnp.float32),
    )(a0, a1, y2, dinv_col, b2)


def kernel(edge_index, x_all, ix, max_iter, W1, b1, W2, b2):
    del ix, max_iter
    src = edge_index[0]
    dst = edge_index[1]
    pad = E_PAD - E
    src_p = jnp.concatenate([src, jnp.zeros((pad,), jnp.int32)])
    dst_p = jnp.concatenate(
        [dst, N + (jnp.arange(pad, dtype=jnp.int32) % (NPAD - N))]
    )
    src2d = jnp.concatenate(
        [src_p.reshape(NW, EPT), jnp.zeros((NW, 4 * CH), jnp.int32)], axis=1
    ).reshape(-1)
    dst2d = jnp.concatenate(
        [dst_p.reshape(NW, EPT),
         jnp.full((NW, 2 * CH), N, dtype=jnp.int32)], axis=1
    ).reshape(-1)

    hist = _sc_hist(dst2d)
    dinv_row = _tc_dinv(hist)
    dinv_col = dinv_row.reshape(NPAD, 1)[:N]
    y = _tc_scale(dinv_col, x_all)

    agg1 = _sc_agg(src2d, dst2d, y)
    y2 = _tc_mid(agg1[0, :N], agg1[1, :N], y, dinv_col, W1, b1, W2)
    agg2 = _sc_agg(src2d, dst2d, y2)
    return _tc_final(agg2[0, :N], agg2[1, :N], y2, dinv_col, b2)


# speedup vs baseline: 2.9257x; 2.9257x over previous
import functools

import jax
import jax.numpy as jnp
from jax import lax
from jax.experimental import pallas as pl
from jax.experimental.pallas import tpu as pltpu
from jax.experimental.pallas import tpu_sc as plsc

N = 10000
E = 320000
D = 128

NC = 2
NS = 16
NW = NC * NS
CH = 128
CPT = 80
EPT = CH * CPT
E_PAD = NW * EPT
NPAD = 10240
ROWS_PT = NPAD // NS
RB = 1000

_mesh = plsc.VectorSubcoreMesh(core_axis_name="c", subcore_axis_name="s")


@functools.partial(
    pl.kernel,
    out_type=jax.ShapeDtypeStruct((NW, NPAD), jnp.float32),
    mesh=_mesh,
    compiler_params=pltpu.CompilerParams(needs_layout_passes=False),
    scratch_types=[
        pltpu.VMEM((CPT, CH), jnp.int32),
        pltpu.VMEM((NPAD,), jnp.float32),
    ],
)
def _sc_hist(dst_hbm, out_hbm, idx_v, hist_v):
    c = lax.axis_index("c")
    s = lax.axis_index("s")
    wid = c * NS + s
    pltpu.sync_copy(dst_hbm.at[wid], idx_v)

    zeros16 = jnp.zeros((16,), jnp.float32)

    def zbody(i, carry):
        hist_v[pl.ds(i * 16, 16)] = zeros16
        return carry

    lax.fori_loop(0, NPAD // 16, zbody, 0)

    ones16 = jnp.ones((16,), jnp.float32)

    def body(j, carry):
        for sub in range(CH // 16):
            idx16 = idx_v[j, pl.ds(sub * 16, 16)]
            plsc.addupdate_scatter(hist_v, [idx16], ones16)
        return carry

    lax.fori_loop(0, CPT, body, 0)
    pltpu.sync_copy(hist_v, out_hbm.at[wid])


@functools.partial(
    pl.kernel,
    out_type=jax.ShapeDtypeStruct((NC, NPAD, D), jnp.float32),
    mesh=_mesh,
    compiler_params=pltpu.CompilerParams(needs_layout_passes=False),
    scratch_types=[
        pltpu.VMEM((CPT, CH), jnp.int32),
        pltpu.VMEM((CPT, CH), jnp.int32),
        pltpu.VMEM((CH,), jnp.int32),
        pltpu.VMEM((CH, D), jnp.float32),
        pltpu.VMEM_SHARED((NPAD, D), jnp.float32),
        pltpu.SemaphoreType.DMA,
    ],
)
def _sc_agg(src_hbm, dst_hbm, y_hbm, out_hbm, src_v, dst_v, dbuf, rows, acc, sem):
    c = lax.axis_index("c")
    s = lax.axis_index("s")
    wid = c * NS + s
    pltpu.sync_copy(src_hbm.at[wid], src_v)
    pltpu.sync_copy(dst_hbm.at[wid], dst_v)

    zeros16 = jnp.zeros((16,), jnp.float32)

    def zbody(i, carry):
        for sub in range(D // 16):
            rows[i, pl.ds(sub * 16, 16)] = zeros16
        return carry

    lax.fori_loop(0, CH, zbody, 0)

    for k in range(ROWS_PT // CH):
        pltpu.sync_copy(rows, acc.at[pl.ds(s * ROWS_PT + k * CH, CH)])
    plsc.subcore_barrier()

    def body(j, carry):
        pltpu.async_copy(y_hbm.at[src_v.at[j]], rows, sem).wait()
        for sub in range(CH // 16):
            dbuf[pl.ds(sub * 16, 16)] = dst_v[j, pl.ds(sub * 16, 16)]
        pltpu.sync_copy(rows, acc.at[dbuf], add=True)
        return carry

    lax.fori_loop(0, CPT, body, 0)
    plsc.subcore_barrier()

    for k in range(ROWS_PT // CH):
        r = s * ROWS_PT + k * CH
        pltpu.sync_copy(acc.at[pl.ds(r, CH)], out_hbm.at[c, pl.ds(r, CH)])


def _tc_dinv(hist):
    def k(h_ref, o_ref):
        deg = jnp.sum(h_ref[...], axis=0, keepdims=True) + 1.0
        o_ref[...] = lax.rsqrt(deg)

    return pl.pallas_call(
        k, out_shape=jax.ShapeDtypeStruct((1, NPAD), jnp.float32)
    )(hist)


def _tc_scale(dinv_col, x):
    def k(d_ref, x_ref, o_ref):
        o_ref[...] = d_ref[...] * x_ref[...]

    return pl.pallas_call(
        k,
        grid=(N // RB,),
        in_specs=[
            pl.BlockSpec((RB, 1), lambda i: (i, 0)),
            pl.BlockSpec((RB, D), lambda i: (i, 0)),
        ],
        out_specs=pl.BlockSpec((RB, D), lambda i: (i, 0)),
        out_shape=jax.ShapeDtypeStruct((N, D), jnp.float32),
    )(dinv_col, x)


def _tc_mid(a0, a1, y, dinv_col, W1, b1, W2):
    def k(a0r, a1r, yr, dr, w1r, b1r, w2r, outr):
        s1 = dr[...] * (a0r[...] + a1r[...] + yr[...])
        x1 = jnp.dot(s1, w1r[...], preferred_element_type=jnp.float32) + b1r[...]
        z = jnp.maximum(x1, 0.0)
        outr[...] = dr[...] * jnp.dot(z, w2r[...], preferred_element_type=jnp.float32)

    row = lambda i: (i, 0)
    fix = lambda i: (0, 0)
    return pl.pallas_call(
        k,
        grid=(N // RB,),
        in_specs=[
            pl.BlockSpec((RB, D), row),
            pl.BlockSpec((RB, D), row),
            pl.BlockSpec((RB, D), row),
            pl.BlockSpec((RB, 1), row),
            pl.BlockSpec((D, 4 * D), fix),
            pl.BlockSpec((4 * D,), lambda i: (0,)),
            pl.BlockSpec((4 * D, D), fix),
        ],
        out_specs=pl.BlockSpec((RB, D), row),
        out_shape=jax.ShapeDtypeStruct((N, D), jnp.float32),
    )(a0, a1, y, dinv_col, W1, b1, W2)


def _tc_final(a0, a1, y2, dinv_col, b2):
    def k(a0r, a1r, yr, dr, b2r, outr):
        outr[...] = dr[...] * (a0r[...] + a1r[...] + yr[...]) + b2r[...]

    row = lambda i: (i, 0)
    return pl.pallas_call(
        k,
        grid=(N // RB,),
        in_specs=[
            pl.BlockSpec((RB, D), row),
            pl.BlockSpec((RB, D), row),
            pl.BlockSpec((RB, D), row),
            pl.BlockSpec((RB, 1), row),
            pl.BlockSpec((D,), lambda i: (0,)),
        ],
        out_specs=pl.BlockSpec((RB, D), row),
        out_shape=jax.ShapeDtypeStruct((N, D), jnp.float32),
    )(a0, a1, y2, dinv_col, b2)


def kernel(edge_index, x_all, ix, max_iter, W1, b1, W2, b2):
    del ix, max_iter
    src = edge_index[0]
    dst = edge_index[1]
    pad = E_PAD - E
    src_p = jnp.concatenate([src, jnp.zeros((pad,), jnp.int32)])
    dst_p = jnp.concatenate(
        [dst, N + (jnp.arange(pad, dtype=jnp.int32) % (NPAD - N))]
    )
    src3d = src_p.reshape(NW, CPT, CH)
    dst3d = dst_p.reshape(NW, CPT, CH)

    hist = _sc_hist(dst3d)
    dinv_row = _tc_dinv(hist)
    dinv_col = dinv_row.reshape(NPAD, 1)[:N]
    y = _tc_scale(dinv_col, x_all)

    agg1 = _sc_agg(src3d, dst3d, y)
    y2 = _tc_mid(agg1[0, :N], agg1[1, :N], y, dinv_col, W1, b1, W2)
    agg2 = _sc_agg(src3d, dst3d, y2)
    return _tc_final(agg2[0, :N], agg2[1, :N], y2, dinv_col, b2)
